# baseline (speedup 1.0000x reference)
import functools

import numpy as np
import jax
import jax.numpy as jnp
from jax import lax
from jax.experimental import pallas as pl
from jax.experimental.pallas import tpu as pltpu
from jax.experimental.pallas import tpu_sc as plsc

_EMBED = 256
_H = 8
_L = 4
_P = 16
_HD = 32
_BS = 4
_LQ = 1024
_SS = np.array([[64, 64], [32, 32], [16, 16], [8, 8]], dtype=np.int64)
_LV = int((_SS[:, 0] * _SS[:, 1]).sum())
_NP_L = [(int(h) + 2) * (int(w) + 2) for h, w in _SS]
_PAD_ROWS = int(sum(_NP_L))
_BASE_L = np.concatenate([[0], np.cumsum(_NP_L)[:-1]]).astype(np.int64)
_SAMP = _L * _P
_TP = _H * _L * _P
_NQ = _BS * _LQ
_NW = 32
_QPW = _NQ // _NW
_QT = 64
_PREC = lax.Precision.HIGHEST


def _build_consts():
    s = np.arange(_TP)
    lvl = (s // _P) % _L
    head = s // (_L * _P)
    gw = _SS[lvl, 1].astype(np.float32)
    gh = _SS[lvl, 0].astype(np.float32)
    stride = gw + 2.0
    sbase = (head * _PAD_ROWS + _BASE_L[lvl]).astype(np.float32)
    return np.stack([gw, gh, stride, sbase]).astype(np.float32)


def _build_sel():
    sel = np.zeros((16, 4 * _TP), np.float32)
    s = np.arange(_TP)
    lvl = (s // _P) % _L
    for comp in range(4):
        sel[lvl * 4 + comp, comp * _TP + s] = 1.0
    return sel


_CONSTS = _build_consts()
_SEL = _build_sel()


def _vproj_body(val_ref, w_ref, b_ref, out_ref):
    z = (jnp.dot(val_ref[0], w_ref[...], preferred_element_type=jnp.float32,
                 precision=_PREC)
         + b_ref[...])
    for h in range(_H):
        out_ref[0, h] = z[:, h * _HD:(h + 1) * _HD]


def _vproj(value, W_val, b_val):
    rows = 680
    return pl.pallas_call(
        _vproj_body,
        grid=(_BS, _LV // rows),
        in_specs=[
            pl.BlockSpec((1, rows, _EMBED), lambda b, t: (b, t, 0)),
            pl.BlockSpec((_EMBED, _EMBED), lambda b, t: (0, 0)),
            pl.BlockSpec((1, _EMBED), lambda b, t: (0, 0)),
        ],
        out_specs=pl.BlockSpec((1, _H, rows, _HD), lambda b, t: (b, 0, t, 0)),
        out_shape=jax.ShapeDtypeStruct((_BS, _H, _LV, _HD), jnp.float32),
    )(value, W_val, b_val.reshape(1, _EMBED))


def _prep_body(q_ref, rp_ref, wq_ref, bq_ref, sel_ref, cons_ref,
               sloc_ref, aw_ref, idx_ref, cw_ref):
    b = pl.program_id(0)
    q = q_ref[0]
    rp = rp_ref[0]
    z = (jnp.dot(q, wq_ref[...], preferred_element_type=jnp.float32,
                 precision=_PREC) + bq_ref[...])
    offx = z[:, :_TP]
    offy = z[:, _TP:2 * _TP]
    logits = z[:, 2 * _TP:]

    rsel = jnp.dot(rp, sel_ref[...], preferred_element_type=jnp.float32,
                   precision=_PREC)
    cx = rsel[:, :_TP]
    cy = rsel[:, _TP:2 * _TP]
    rw = rsel[:, 2 * _TP:3 * _TP]
    rh = rsel[:, 3 * _TP:]

    l3 = logits.reshape(_QT, _H, _SAMP)
    m = jnp.max(l3, axis=-1, keepdims=True)
    e = jnp.exp(l3 - m)
    aw = (e / jnp.sum(e, axis=-1, keepdims=True)).reshape(_QT, _TP)

    gw = cons_ref[0, :].reshape(1, _TP)
    gh = cons_ref[1, :].reshape(1, _TP)
    stride = cons_ref[2, :].reshape(1, _TP)
    sbase = cons_ref[3, :].reshape(1, _TP)

    scale = 0.5 / _P
    sx = cx + offx * (rw * scale)
    sy = cy + offy * (rh * scale)

    x = sx * gw - 0.5
    y = sy * gh - 0.5
    x0 = jnp.floor(x)
    y0 = jnp.floor(y)
    fx = x - x0
    fy = y - y0
    one = jnp.float32(1.0)
    vx0 = ((x0 >= 0) & (x0 <= gw - 1)).astype(jnp.float32)
    vx1 = ((x0 >= -1) & (x0 <= gw - 2)).astype(jnp.float32)
    vy0 = ((y0 >= 0) & (y0 <= gh - 1)).astype(jnp.float32)
    vy1 = ((y0 >= -1) & (y0 <= gh - 2)).astype(jnp.float32)

    cw00 = aw * (one - fx) * (one - fy) * vx0 * vy0
    cw01 = aw * fx * (one - fy) * vx1 * vy0
    cw10 = aw * (one - fx) * fy * vx0 * vy1
    cw11 = aw * fx * fy * vx1 * vy1

    x0c = jnp.clip(x0, -1.0, gw - 1)
    y0c = jnp.clip(y0, -1.0, gh - 1)
    ilocal = (y0c + 1.0) * stride + (x0c + 1.0)
    idx = (sbase + ilocal).astype(jnp.int32) + b * (_H * _PAD_ROWS)

    sloc_ref[0] = jnp.stack([sx, sy], axis=-1).reshape(_QT, 2 * _TP)
    aw_ref[0] = aw
    idx_ref[0] = idx
    cw_ref[0] = jnp.stack([cw00, cw01, cw10, cw11],
                          axis=-1).reshape(_QT, 4 * _TP)


def _prep(query, rp16, WQ, bq):
    grid = (_BS, _LQ // _QT)
    return pl.pallas_call(
        _prep_body,
        grid=grid,
        in_specs=[
            pl.BlockSpec((1, _QT, _EMBED), lambda b, t: (b, t, 0)),
            pl.BlockSpec((1, _QT, 16), lambda b, t: (b, t, 0)),
            pl.BlockSpec((_EMBED, 3 * _TP), lambda b, t: (0, 0)),
            pl.BlockSpec((1, 3 * _TP), lambda b, t: (0, 0)),
            pl.BlockSpec((16, 4 * _TP), lambda b, t: (0, 0)),
            pl.BlockSpec((4, _TP), lambda b, t: (0, 0)),
        ],
        out_specs=[
            pl.BlockSpec((1, _QT, 2 * _TP), lambda b, t: (b, t, 0)),
            pl.BlockSpec((1, _QT, _TP), lambda b, t: (b, t, 0)),
            pl.BlockSpec((1, _QT, _TP), lambda b, t: (b, t, 0)),
            pl.BlockSpec((1, _QT, 4 * _TP), lambda b, t: (b, t, 0)),
        ],
        out_shape=[
            jax.ShapeDtypeStruct((_BS, _LQ, 2 * _TP), jnp.float32),
            jax.ShapeDtypeStruct((_BS, _LQ, _TP), jnp.float32),
            jax.ShapeDtypeStruct((_BS, _LQ, _TP), jnp.int32),
            jax.ShapeDtypeStruct((_BS, _LQ, 4 * _TP), jnp.float32),
        ],
    )(query, rp16, WQ, bq.reshape(1, 3 * _TP),
      jnp.asarray(_SEL), jnp.asarray(_CONSTS))


def _sc_body(table, idxs, cws, out,
             idxA, idxB, cwA, cwB, rows0, rows1, rows2, rows3, outA, outB,
             g0, g1, g2, g3, oA, oB, iiA, iiB, icA, icB):
    cid = lax.axis_index("c")
    sid = lax.axis_index("s")
    wid = sid * 2 + cid
    qbase = wid * _QPW
    idxb = (idxA, idxB)
    cwb = (cwA, cwB)
    outb = (outA, outB)
    rows = (rows0, rows1, rows2, rows3)
    gsem = (g0, g1, g2, g3)
    osem = (oA, oB)
    iisem = (iiA, iiB)
    icsem = (icA, icB)

    pltpu.sync_copy(idxs.at[qbase], idxA)
    pltpu.sync_copy(cws.at[qbase], cwA)
    pltpu.async_copy(idxs.at[qbase + 1], idxB, iiB)
    pltpu.async_copy(cws.at[qbase + 1], cwB, icB)
    pltpu.async_copy(table.at[idxA.at[0]], rows0, g0)
    pltpu.async_copy(table.at[idxA.at[1]], rows1, g1)

    def pair_body(i, carry):
        for par in range(2):
            q = qbase + 2 * i + par
            idx_v = idxb[par]
            cw_v = cwb[par]
            out_v = outb[par]
            oth = 1 - par
            for h in range(_H):
                if h == 0:
                    @pl.when(i > 0)
                    def _():
                        pltpu.make_async_copy(out_v, out.at[q],
                                              osem[par]).wait()
                if h == 6:
                    pltpu.make_async_copy(idxs.at[q], idxb[oth],
                                          iisem[oth]).wait()
                    pltpu.make_async_copy(cws.at[q], cwb[oth],
                                          icsem[oth]).wait()
                if h < 6:
                    nsrc = idx_v.at[h + 2]
                else:
                    nsrc = idxb[oth].at[h - 6]
                nb = (h + 2) % 4
                pltpu.async_copy(table.at[nsrc], rows[nb], gsem[nb])
                pltpu.make_async_copy(table.at[idx_v.at[0]], rows[h % 4],
                                      gsem[h % 4]).wait()
                buf = rows[h % 4]

                def s_body(jj, acc):
                    a0, a1 = acc
                    wv = cw_v[h, jj]
                    for k in range(4):
                        j = jj * 4 + k
                        a0 = a0 + wv[4 * k] * buf[j, pl.ds(0, 16)]
                        a1 = a1 + wv[4 * k] * buf[j, pl.ds(16, 16)]
                        a0 = a0 + wv[4 * k + 1] * buf[j, pl.ds(32, 16)]
                        a1 = a1 + wv[4 * k + 1] * buf[j, pl.ds(48, 16)]
                        a0 = a0 + wv[4 * k + 2] * buf[j, pl.ds(64, 16)]
                        a1 = a1 + wv[4 * k + 2] * buf[j, pl.ds(80, 16)]
                        a0 = a0 + wv[4 * k + 3] * buf[j, pl.ds(96, 16)]
                        a1 = a1 + wv[4 * k + 3] * buf[j, pl.ds(112, 16)]
                    return (a0, a1)

                zero = jnp.zeros((16,), jnp.float32)
                a0, a1 = lax.fori_loop(0, _SAMP // 4, s_body, (zero, zero))
                out_v[pl.ds(h * 32, 16)] = a0
                out_v[pl.ds(h * 32 + 16, 16)] = a1
            pltpu.async_copy(out_v, out.at[q], osem[par])
            qn = jnp.minimum(q + 2, _NQ - 1)
            pltpu.async_copy(idxs.at[qn], idx_v, iisem[par])
            pltpu.async_copy(cws.at[qn], cw_v, icsem[par])
        return carry

    lax.fori_loop(0, _QPW // 2, pair_body, 0)

    pltpu.make_async_copy(table.at[idxA.at[0]], rows0, g0).wait()
    pltpu.make_async_copy(table.at[idxA.at[0]], rows1, g1).wait()
    pltpu.make_async_copy(idxs.at[qbase], idxB, iiB).wait()
    pltpu.make_async_copy(cws.at[qbase], cwB, icB).wait()
    pltpu.make_async_copy(outA, out.at[qbase], oA).wait()
    pltpu.make_async_copy(outB, out.at[qbase], oB).wait()


def _sc_gather(table, idxs, cws):
    mesh = plsc.VectorSubcoreMesh(core_axis_name="c", subcore_axis_name="s",
                                  num_cores=2, num_subcores=16)
    return pl.kernel(
        _sc_body,
        out_type=jax.ShapeDtypeStruct((_NQ, _EMBED), jnp.float32),
        mesh=mesh,
        scratch_types=[
            pltpu.VMEM((_H, _SAMP), jnp.int32),
            pltpu.VMEM((_H, _SAMP), jnp.int32),
            pltpu.VMEM((_H, _SAMP // 4, 16), jnp.float32),
            pltpu.VMEM((_H, _SAMP // 4, 16), jnp.float32),
            pltpu.VMEM((_SAMP, 128), jnp.float32),
            pltpu.VMEM((_SAMP, 128), jnp.float32),
            pltpu.VMEM((_SAMP, 128), jnp.float32),
            pltpu.VMEM((_SAMP, 128), jnp.float32),
            pltpu.VMEM((_EMBED,), jnp.float32),
            pltpu.VMEM((_EMBED,), jnp.float32),
            pltpu.SemaphoreType.DMA,
            pltpu.SemaphoreType.DMA,
            pltpu.SemaphoreType.DMA,
            pltpu.SemaphoreType.DMA,
            pltpu.SemaphoreType.DMA,
            pltpu.SemaphoreType.DMA,
            pltpu.SemaphoreType.DMA,
            pltpu.SemaphoreType.DMA,
            pltpu.SemaphoreType.DMA,
            pltpu.SemaphoreType.DMA,
        ],
    )(table, idxs, cws)


def _outproj_body(x_ref, w_ref, b_ref, out_ref):
    out_ref[...] = (
        jnp.dot(x_ref[...], w_ref[...], preferred_element_type=jnp.float32,
                precision=_PREC)
        + b_ref[...]
    )


def _outproj(x, W_out, b_out):
    rows = 512
    return pl.pallas_call(
        _outproj_body,
        grid=(_NQ // rows,),
        in_specs=[
            pl.BlockSpec((rows, _EMBED), lambda t: (t, 0)),
            pl.BlockSpec((_EMBED, _EMBED), lambda t: (0, 0)),
            pl.BlockSpec((1, _EMBED), lambda t: (0, 0)),
        ],
        out_specs=pl.BlockSpec((rows, _EMBED), lambda t: (t, 0)),
        out_shape=jax.ShapeDtypeStruct((_NQ, _EMBED), jnp.float32),
    )(x, W_out, b_out.reshape(1, _EMBED))


def _build_table(v4):
    parts = []
    off = 0
    for (hh, ww), npl in zip(_SS, _NP_L):
        hh, ww = int(hh), int(ww)
        w2 = ww + 2
        g = v4[:, :, off:off + hh * ww].reshape(_BS, _H, hh, ww, _HD)
        gp = jnp.pad(g, ((0, 0), (0, 0), (1, 1), (1, 1), (0, 0)))
        flat = gp.reshape(_BS, _H, npl, _HD)
        flat = jnp.pad(flat, ((0, 0), (0, 0), (0, w2 + 1), (0, 0)))
        quad = jnp.concatenate([
            flat[:, :, 0:npl],
            flat[:, :, 1:npl + 1],
            flat[:, :, w2:npl + w2],
            flat[:, :, w2 + 1:npl + w2 + 1],
        ], axis=-1)
        parts.append(quad)
        off += hh * ww
    table = jnp.concatenate(parts, axis=2)
    return table.reshape(_BS * _H * _PAD_ROWS, 4 * _HD)


def kernel(query, reference_points, value, value_spatial_shapes, W_off,
           b_off, W_attn, b_attn, W_val, b_val, W_out, b_out):
    v4 = _vproj(value, W_val, b_val)
    table = _build_table(v4)

    Wx = W_off.reshape(_EMBED, _TP, 2)[:, :, 0]
    Wy = W_off.reshape(_EMBED, _TP, 2)[:, :, 1]
    WQ = jnp.concatenate([Wx, Wy, W_attn], axis=1)
    bx = b_off.reshape(_TP, 2)[:, 0]
    by = b_off.reshape(_TP, 2)[:, 1]
    bq = jnp.concatenate([bx, by, b_attn])
    rp16 = reference_points.reshape(_BS, _LQ, 16)
    sloc_flat, aw, idx, cw_flat = _prep(query, rp16, WQ, bq)

    idxs = idx.reshape(_NQ, _H, _SAMP)
    cws = cw_flat.reshape(_NQ, _H, _SAMP // 4, 16)

    core = _sc_gather(table, idxs, cws)

    out = _outproj(core, W_out, b_out).reshape(_BS, _LQ, _EMBED)

    sloc = sloc_flat.reshape(_BS, _LQ, _H, _L, _P, 2)
    aw_out = aw.reshape(_BS, _LQ, _H, _L, _P)
    return (out, sloc, aw_out)


# speedup vs baseline: 2.7888x; 2.7888x over previous
import functools

import numpy as np
import jax
import jax.numpy as jnp
from jax import lax
from jax.experimental import pallas as pl
from jax.experimental.pallas import tpu as pltpu
from jax.experimental.pallas import tpu_sc as plsc

_EMBED = 256
_H = 8
_L = 4
_P = 16
_HD = 32
_BS = 4
_LQ = 1024
_SS = np.array([[64, 64], [32, 32], [16, 16], [8, 8]], dtype=np.int64)
_LV = int((_SS[:, 0] * _SS[:, 1]).sum())
_NP_L = [(int(h) + 2) * (int(w) + 2) for h, w in _SS]
_PAD_ROWS = int(sum(_NP_L))
_BASE_L = np.concatenate([[0], np.cumsum(_NP_L)[:-1]]).astype(np.int64)
_SAMP = _L * _P
_TP = _H * _L * _P
_NQ = _BS * _LQ
_NW = 32
_QPW = _NQ // _NW
_QT = 128
_PREC = lax.Precision.HIGHEST


def _build_consts():
    s = np.arange(_TP)
    lvl = (s // _P) % _L
    head = s // (_L * _P)
    gw = _SS[lvl, 1].astype(np.float32)
    gh = _SS[lvl, 0].astype(np.float32)
    stride = gw + 2.0
    sbase = (head * _PAD_ROWS + _BASE_L[lvl]).astype(np.float32)
    return np.stack([gw, gh, stride, sbase]).astype(np.float32)


def _build_sel():
    sel = np.zeros((16, 4 * _TP), np.float32)
    s = np.arange(_TP)
    lvl = (s // _P) % _L
    for comp in range(4):
        sel[lvl * 4 + comp, comp * _TP + s] = 1.0
    return sel


_CONSTS = _build_consts()
_SEL = _build_sel()


def _vproj_body(val_ref, w_ref, b_ref, out_ref):
    z = (jnp.dot(val_ref[0], w_ref[...], preferred_element_type=jnp.float32,
                 precision=_PREC)
         + b_ref[...])
    for h in range(_H):
        out_ref[0, h] = z[:, h * _HD:(h + 1) * _HD]


def _vproj(value, W_val, b_val):
    rows = 680
    return pl.pallas_call(
        _vproj_body,
        grid=(_BS, _LV // rows),
        in_specs=[
            pl.BlockSpec((1, rows, _EMBED), lambda b, t: (b, t, 0)),
            pl.BlockSpec((_EMBED, _EMBED), lambda b, t: (0, 0)),
            pl.BlockSpec((1, _EMBED), lambda b, t: (0, 0)),
        ],
        out_specs=pl.BlockSpec((1, _H, rows, _HD), lambda b, t: (b, 0, t, 0)),
        out_shape=jax.ShapeDtypeStruct((_BS, _H, _LV, _HD), jnp.float32),
    )(value, W_val, b_val.reshape(1, _EMBED))


def _prep_body(q_ref, rp_ref, wq_ref, bq_ref, sel_ref, cons_ref,
               sloc_ref, aw_ref, idx_ref, cw_ref):
    b = pl.program_id(0)
    q = q_ref[0]
    rp = rp_ref[0]
    z = (jnp.dot(q, wq_ref[...], preferred_element_type=jnp.float32,
                 precision=_PREC) + bq_ref[...])
    offx = z[:, :_TP]
    offy = z[:, _TP:2 * _TP]
    logits = z[:, 2 * _TP:]

    rsel = jnp.dot(rp, sel_ref[...], preferred_element_type=jnp.float32,
                   precision=_PREC)
    cx = rsel[:, :_TP]
    cy = rsel[:, _TP:2 * _TP]
    rw = rsel[:, 2 * _TP:3 * _TP]
    rh = rsel[:, 3 * _TP:]

    l3 = logits.reshape(_QT, _H, _SAMP)
    m = jnp.max(l3, axis=-1, keepdims=True)
    e = jnp.exp(l3 - m)
    aw = (e / jnp.sum(e, axis=-1, keepdims=True)).reshape(_QT, _TP)

    gw = cons_ref[0, :].reshape(1, _TP)
    gh = cons_ref[1, :].reshape(1, _TP)
    stride = cons_ref[2, :].reshape(1, _TP)
    sbase = cons_ref[3, :].reshape(1, _TP)

    scale = 0.5 / _P
    sx = cx + offx * (rw * scale)
    sy = cy + offy * (rh * scale)

    x = sx * gw - 0.5
    y = sy * gh - 0.5
    x0 = jnp.floor(x)
    y0 = jnp.floor(y)
    fx = x - x0
    fy = y - y0
    one = jnp.float32(1.0)
    vx0 = ((x0 >= 0) & (x0 <= gw - 1)).astype(jnp.float32)
    vx1 = ((x0 >= -1) & (x0 <= gw - 2)).astype(jnp.float32)
    vy0 = ((y0 >= 0) & (y0 <= gh - 1)).astype(jnp.float32)
    vy1 = ((y0 >= -1) & (y0 <= gh - 2)).astype(jnp.float32)

    cw00 = aw * (one - fx) * (one - fy) * vx0 * vy0
    cw01 = aw * fx * (one - fy) * vx1 * vy0
    cw10 = aw * (one - fx) * fy * vx0 * vy1
    cw11 = aw * fx * fy * vx1 * vy1

    x0c = jnp.clip(x0, -1.0, gw - 1)
    y0c = jnp.clip(y0, -1.0, gh - 1)
    ilocal = (y0c + 1.0) * stride + (x0c + 1.0)
    idx = (sbase + ilocal).astype(jnp.int32) + b * (_H * _PAD_ROWS)

    sloc_ref[0, 0] = sx
    sloc_ref[0, 1] = sy
    aw_ref[0] = aw
    idx_ref[0] = idx
    cw_ref[0, 0] = cw00
    cw_ref[0, 1] = cw01
    cw_ref[0, 2] = cw10
    cw_ref[0, 3] = cw11


def _prep(query, rp16, WQ, bq):
    grid = (_BS, _LQ // _QT)
    return pl.pallas_call(
        _prep_body,
        grid=grid,
        in_specs=[
            pl.BlockSpec((1, _QT, _EMBED), lambda b, t: (b, t, 0)),
            pl.BlockSpec((1, _QT, 16), lambda b, t: (b, t, 0)),
            pl.BlockSpec((_EMBED, 3 * _TP), lambda b, t: (0, 0)),
            pl.BlockSpec((1, 3 * _TP), lambda b, t: (0, 0)),
            pl.BlockSpec((16, 4 * _TP), lambda b, t: (0, 0)),
            pl.BlockSpec((4, _TP), lambda b, t: (0, 0)),
        ],
        out_specs=[
            pl.BlockSpec((1, 2, _QT, _TP), lambda b, t: (b, 0, t, 0)),
            pl.BlockSpec((1, _QT, _TP), lambda b, t: (b, t, 0)),
            pl.BlockSpec((1, _QT, _TP), lambda b, t: (b, t, 0)),
            pl.BlockSpec((1, 4, _QT, _TP), lambda b, t: (b, 0, t, 0)),
        ],
        out_shape=[
            jax.ShapeDtypeStruct((_BS, 2, _LQ, _TP), jnp.float32),
            jax.ShapeDtypeStruct((_BS, _LQ, _TP), jnp.float32),
            jax.ShapeDtypeStruct((_BS, _LQ, _TP), jnp.int32),
            jax.ShapeDtypeStruct((_BS, 4, _LQ, _TP), jnp.float32),
        ],
    )(query, rp16, WQ, bq.reshape(1, 3 * _TP),
      jnp.asarray(_SEL), jnp.asarray(_CONSTS))


def _sc_body(table, idxs, cws, out,
             idxA, idxB, cwA, cwB, rows0, rows1, rows2, rows3, outA, outB,
             g0, g1, g2, g3, oA, oB, iiA, iiB, icA, icB):
    cid = lax.axis_index("c")
    sid = lax.axis_index("s")
    wid = sid * 2 + cid
    qbase = wid * _QPW
    idxb = (idxA, idxB)
    cwb = (cwA, cwB)
    outb = (outA, outB)
    rows = (rows0, rows1, rows2, rows3)
    gsem = (g0, g1, g2, g3)
    osem = (oA, oB)
    iisem = (iiA, iiB)
    icsem = (icA, icB)

    pltpu.sync_copy(idxs.at[qbase], idxA)
    pltpu.sync_copy(cws.at[qbase], cwA)
    pltpu.async_copy(idxs.at[qbase + 1], idxB, iiB)
    pltpu.async_copy(cws.at[qbase + 1], cwB, icB)
    pltpu.async_copy(table.at[idxA.at[0]], rows0, g0)
    pltpu.async_copy(table.at[idxA.at[1]], rows1, g1)

    def pair_body(i, carry):
        for par in range(2):
            q = qbase + 2 * i + par
            idx_v = idxb[par]
            cw_v = cwb[par]
            out_v = outb[par]
            oth = 1 - par
            for h in range(_H):
                if h == 0:
                    @pl.when(i > 0)
                    def _():
                        pltpu.make_async_copy(out_v, out.at[q],
                                              osem[par]).wait()
                if h == 6:
                    pltpu.make_async_copy(idxs.at[q], idxb[oth],
                                          iisem[oth]).wait()
                    pltpu.make_async_copy(cws.at[q], cwb[oth],
                                          icsem[oth]).wait()
                if h < 6:
                    nsrc = idx_v.at[h + 2]
                else:
                    nsrc = idxb[oth].at[h - 6]
                nb = (h + 2) % 4
                pltpu.async_copy(table.at[nsrc], rows[nb], gsem[nb])
                pltpu.make_async_copy(table.at[idx_v.at[0]], rows[h % 4],
                                      gsem[h % 4]).wait()
                buf = rows[h % 4]

                def s_body(jj, acc):
                    a0, a1 = acc
                    wv = cw_v[h, jj]
                    for k in range(4):
                        j = jj * 4 + k
                        a0 = a0 + wv[4 * k] * buf[j, pl.ds(0, 16)]
                        a1 = a1 + wv[4 * k] * buf[j, pl.ds(16, 16)]
                        a0 = a0 + wv[4 * k + 1] * buf[j, pl.ds(32, 16)]
                        a1 = a1 + wv[4 * k + 1] * buf[j, pl.ds(48, 16)]
                        a0 = a0 + wv[4 * k + 2] * buf[j, pl.ds(64, 16)]
                        a1 = a1 + wv[4 * k + 2] * buf[j, pl.ds(80, 16)]
                        a0 = a0 + wv[4 * k + 3] * buf[j, pl.ds(96, 16)]
                        a1 = a1 + wv[4 * k + 3] * buf[j, pl.ds(112, 16)]
                    return (a0, a1)

                zero = jnp.zeros((16,), jnp.float32)
                a0, a1 = lax.fori_loop(0, _SAMP // 4, s_body, (zero, zero))
                out_v[pl.ds(h * 32, 16)] = a0
                out_v[pl.ds(h * 32 + 16, 16)] = a1
            pltpu.async_copy(out_v, out.at[q], osem[par])
            qn = jnp.minimum(q + 2, _NQ - 1)
            pltpu.async_copy(idxs.at[qn], idx_v, iisem[par])
            pltpu.async_copy(cws.at[qn], cw_v, icsem[par])
        return carry

    lax.fori_loop(0, _QPW // 2, pair_body, 0)

    pltpu.make_async_copy(table.at[idxA.at[0]], rows0, g0).wait()
    pltpu.make_async_copy(table.at[idxA.at[0]], rows1, g1).wait()
    pltpu.make_async_copy(idxs.at[qbase], idxB, iiB).wait()
    pltpu.make_async_copy(cws.at[qbase], cwB, icB).wait()
    pltpu.make_async_copy(outA, out.at[qbase], oA).wait()
    pltpu.make_async_copy(outB, out.at[qbase], oB).wait()


def _sc_gather(table, idxs, cws):
    mesh = plsc.VectorSubcoreMesh(core_axis_name="c", subcore_axis_name="s",
                                  num_cores=2, num_subcores=16)
    return pl.kernel(
        _sc_body,
        out_type=jax.ShapeDtypeStruct((_NQ, _EMBED), jnp.float32),
        mesh=mesh,
        scratch_types=[
            pltpu.VMEM((_H, _SAMP), jnp.int32),
            pltpu.VMEM((_H, _SAMP), jnp.int32),
            pltpu.VMEM((_H, _SAMP // 4, 16), jnp.float32),
            pltpu.VMEM((_H, _SAMP // 4, 16), jnp.float32),
            pltpu.VMEM((_SAMP, 128), jnp.float32),
            pltpu.VMEM((_SAMP, 128), jnp.float32),
            pltpu.VMEM((_SAMP, 128), jnp.float32),
            pltpu.VMEM((_SAMP, 128), jnp.float32),
            pltpu.VMEM((_EMBED,), jnp.float32),
            pltpu.VMEM((_EMBED,), jnp.float32),
            pltpu.SemaphoreType.DMA,
            pltpu.SemaphoreType.DMA,
            pltpu.SemaphoreType.DMA,
            pltpu.SemaphoreType.DMA,
            pltpu.SemaphoreType.DMA,
            pltpu.SemaphoreType.DMA,
            pltpu.SemaphoreType.DMA,
            pltpu.SemaphoreType.DMA,
            pltpu.SemaphoreType.DMA,
            pltpu.SemaphoreType.DMA,
        ],
    )(table, idxs, cws)


def _outproj_body(x_ref, w_ref, b_ref, out_ref):
    out_ref[...] = (
        jnp.dot(x_ref[...], w_ref[...], preferred_element_type=jnp.float32,
                precision=_PREC)
        + b_ref[...]
    )


def _outproj(x, W_out, b_out):
    rows = 512
    return pl.pallas_call(
        _outproj_body,
        grid=(_NQ // rows,),
        in_specs=[
            pl.BlockSpec((rows, _EMBED), lambda t: (t, 0)),
            pl.BlockSpec((_EMBED, _EMBED), lambda t: (0, 0)),
            pl.BlockSpec((1, _EMBED), lambda t: (0, 0)),
        ],
        out_specs=pl.BlockSpec((rows, _EMBED), lambda t: (t, 0)),
        out_shape=jax.ShapeDtypeStruct((_NQ, _EMBED), jnp.float32),
    )(x, W_out, b_out.reshape(1, _EMBED))


def _build_table(v4):
    parts = []
    off = 0
    for (hh, ww), npl in zip(_SS, _NP_L):
        hh, ww = int(hh), int(ww)
        w2 = ww + 2
        g = v4[:, :, off:off + hh * ww].reshape(_BS, _H, hh, ww, _HD)
        gp = jnp.pad(g, ((0, 0), (0, 0), (1, 1), (1, 1), (0, 0)))
        flat = gp.reshape(_BS, _H, npl, _HD)
        flat = jnp.pad(flat, ((0, 0), (0, 0), (0, w2 + 1), (0, 0)))
        quad = jnp.concatenate([
            flat[:, :, 0:npl],
            flat[:, :, 1:npl + 1],
            flat[:, :, w2:npl + w2],
            flat[:, :, w2 + 1:npl + w2 + 1],
        ], axis=-1)
        parts.append(quad)
        off += hh * ww
    table = jnp.concatenate(parts, axis=2)
    return table.reshape(_BS * _H * _PAD_ROWS, 4 * _HD)


def kernel(query, reference_points, value, value_spatial_shapes, W_off,
           b_off, W_attn, b_attn, W_val, b_val, W_out, b_out):
    v4 = _vproj(value, W_val, b_val)
    table = _build_table(v4)

    Wx = W_off.reshape(_EMBED, _TP, 2)[:, :, 0]
    Wy = W_off.reshape(_EMBED, _TP, 2)[:, :, 1]
    WQ = jnp.concatenate([Wx, Wy, W_attn], axis=1)
    bx = b_off.reshape(_TP, 2)[:, 0]
    by = b_off.reshape(_TP, 2)[:, 1]
    bq = jnp.concatenate([bx, by, b_attn])
    rp16 = reference_points.reshape(_BS, _LQ, 16)
    sloc2, aw, idx, cw4 = _prep(query, rp16, WQ, bq)

    idxs = idx.reshape(_NQ, _H, _SAMP)
    cws = jnp.moveaxis(cw4, 1, -1).reshape(_NQ, _H, _SAMP // 4, 16)

    core = _sc_gather(table, idxs, cws)

    out = _outproj(core, W_out, b_out).reshape(_BS, _LQ, _EMBED)

    sloc = jnp.stack([sloc2[:, 0], sloc2[:, 1]], axis=-1)
    sloc = sloc.reshape(_BS, _LQ, _H, _L, _P, 2)
    aw_out = aw.reshape(_BS, _LQ, _H, _L, _P)
    return (out, sloc, aw_out)


# speedup vs baseline: 2.7924x; 1.0013x over previous
import functools

import numpy as np
import jax
import jax.numpy as jnp
from jax import lax
from jax.experimental import pallas as pl
from jax.experimental.pallas import tpu as pltpu
from jax.experimental.pallas import tpu_sc as plsc

_EMBED = 256
_H = 8
_L = 4
_P = 16
_HD = 32
_BS = 4
_LQ = 1024
_SS = np.array([[64, 64], [32, 32], [16, 16], [8, 8]], dtype=np.int64)
_LV = int((_SS[:, 0] * _SS[:, 1]).sum())
_NP_L = [(int(h) + 2) * (int(w) + 2) for h, w in _SS]
_PAD_ROWS = int(sum(_NP_L))
_BASE_L = np.concatenate([[0], np.cumsum(_NP_L)[:-1]]).astype(np.int64)
_SAMP = _L * _P
_TP = _H * _L * _P
_NQ = _BS * _LQ
_NW = 32
_QPW = _NQ // _NW
_QT = 128
_PREC = lax.Precision.HIGHEST


def _build_consts():
    s = np.arange(_TP)
    lvl = (s // _P) % _L
    head = s // (_L * _P)
    gw = _SS[lvl, 1].astype(np.float32)
    gh = _SS[lvl, 0].astype(np.float32)
    stride = gw + 2.0
    sbase = (head * _PAD_ROWS + _BASE_L[lvl]).astype(np.float32)
    return np.stack([gw, gh, stride, sbase]).astype(np.float32)


def _build_sel():
    sel = np.zeros((16, 4 * _TP), np.float32)
    s = np.arange(_TP)
    lvl = (s // _P) % _L
    for comp in range(4):
        sel[lvl * 4 + comp, comp * _TP + s] = 1.0
    return sel


_CONSTS = _build_consts()
_SEL = _build_sel()


def _vproj_body(val_ref, w_ref, b_ref, out_ref):
    z = (jnp.dot(val_ref[0], w_ref[...], preferred_element_type=jnp.float32,
                 precision=_PREC)
         + b_ref[...])
    for h in range(_H):
        out_ref[0, h] = z[:, h * _HD:(h + 1) * _HD]


def _vproj(value, W_val, b_val):
    rows = 680
    return pl.pallas_call(
        _vproj_body,
        grid=(_BS, _LV // rows),
        in_specs=[
            pl.BlockSpec((1, rows, _EMBED), lambda b, t: (b, t, 0)),
            pl.BlockSpec((_EMBED, _EMBED), lambda b, t: (0, 0)),
            pl.BlockSpec((1, _EMBED), lambda b, t: (0, 0)),
        ],
        out_specs=pl.BlockSpec((1, _H, rows, _HD), lambda b, t: (b, 0, t, 0)),
        out_shape=jax.ShapeDtypeStruct((_BS, _H, _LV, _HD), jnp.float32),
    )(value, W_val, b_val.reshape(1, _EMBED))


def _prep_body(q_ref, rp_ref, wq_ref, bq_ref, sel_ref, cons_ref,
               sloc_ref, aw_ref, idx_ref, cw_ref):
    b = pl.program_id(0)
    q = q_ref[0]
    rp = rp_ref[0]
    z = (jnp.dot(q, wq_ref[...], preferred_element_type=jnp.float32,
                 precision=_PREC) + bq_ref[...])
    offx = z[:, :_TP]
    offy = z[:, _TP:2 * _TP]
    logits = z[:, 2 * _TP:]

    rsel = jnp.dot(rp, sel_ref[...], preferred_element_type=jnp.float32,
                   precision=_PREC)
    cx = rsel[:, :_TP]
    cy = rsel[:, _TP:2 * _TP]
    rw = rsel[:, 2 * _TP:3 * _TP]
    rh = rsel[:, 3 * _TP:]

    l3 = logits.reshape(_QT, _H, _SAMP)
    m = jnp.max(l3, axis=-1, keepdims=True)
    e = jnp.exp(l3 - m)
    aw = (e / jnp.sum(e, axis=-1, keepdims=True)).reshape(_QT, _TP)

    gw = cons_ref[0, :].reshape(1, _TP)
    gh = cons_ref[1, :].reshape(1, _TP)
    stride = cons_ref[2, :].reshape(1, _TP)
    sbase = cons_ref[3, :].reshape(1, _TP)

    scale = 0.5 / _P
    sx = cx + offx * (rw * scale)
    sy = cy + offy * (rh * scale)

    x = sx * gw - 0.5
    y = sy * gh - 0.5
    x0 = jnp.floor(x)
    y0 = jnp.floor(y)
    fx = x - x0
    fy = y - y0
    one = jnp.float32(1.0)
    vx0 = ((x0 >= 0) & (x0 <= gw - 1)).astype(jnp.float32)
    vx1 = ((x0 >= -1) & (x0 <= gw - 2)).astype(jnp.float32)
    vy0 = ((y0 >= 0) & (y0 <= gh - 1)).astype(jnp.float32)
    vy1 = ((y0 >= -1) & (y0 <= gh - 2)).astype(jnp.float32)

    cw00 = aw * (one - fx) * (one - fy) * vx0 * vy0
    cw01 = aw * fx * (one - fy) * vx1 * vy0
    cw10 = aw * (one - fx) * fy * vx0 * vy1
    cw11 = aw * fx * fy * vx1 * vy1

    x0c = jnp.clip(x0, -1.0, gw - 1)
    y0c = jnp.clip(y0, -1.0, gh - 1)
    ilocal = (y0c + 1.0) * stride + (x0c + 1.0)
    idx = (sbase + ilocal).astype(jnp.int32) + b * (_H * _PAD_ROWS)

    sloc_ref[0, 0] = sx
    sloc_ref[0, 1] = sy
    aw_ref[0] = aw
    idx_ref[0] = idx
    cw_ref[0, 0] = cw00
    cw_ref[0, 1] = cw01
    cw_ref[0, 2] = cw10
    cw_ref[0, 3] = cw11


def _prep(query, rp16, WQ, bq):
    grid = (_BS, _LQ // _QT)
    return pl.pallas_call(
        _prep_body,
        grid=grid,
        in_specs=[
            pl.BlockSpec((1, _QT, _EMBED), lambda b, t: (b, t, 0)),
            pl.BlockSpec((1, _QT, 16), lambda b, t: (b, t, 0)),
            pl.BlockSpec((_EMBED, 3 * _TP), lambda b, t: (0, 0)),
            pl.BlockSpec((1, 3 * _TP), lambda b, t: (0, 0)),
            pl.BlockSpec((16, 4 * _TP), lambda b, t: (0, 0)),
            pl.BlockSpec((4, _TP), lambda b, t: (0, 0)),
        ],
        out_specs=[
            pl.BlockSpec((1, 2, _QT, _TP), lambda b, t: (b, 0, t, 0)),
            pl.BlockSpec((1, _QT, _TP), lambda b, t: (b, t, 0)),
            pl.BlockSpec((1, _QT, _TP), lambda b, t: (b, t, 0)),
            pl.BlockSpec((1, 4, _QT, _TP), lambda b, t: (b, 0, t, 0)),
        ],
        out_shape=[
            jax.ShapeDtypeStruct((_BS, 2, _LQ, _TP), jnp.float32),
            jax.ShapeDtypeStruct((_BS, _LQ, _TP), jnp.float32),
            jax.ShapeDtypeStruct((_BS, _LQ, _TP), jnp.int32),
            jax.ShapeDtypeStruct((_BS, 4, _LQ, _TP), jnp.float32),
        ],
    )(query, rp16, WQ, bq.reshape(1, 3 * _TP),
      jnp.asarray(_SEL), jnp.asarray(_CONSTS))


def _sc_body(table, idxs, cws, out,
             idxA, idxB, cwA, cwB, rows0, rows1, rows2, rows3, outA, outB,
             g0, g1, g2, g3, oA, oB, iiA, iiB, icA, icB):
    cid = lax.axis_index("c")
    sid = lax.axis_index("s")
    wid = sid * 2 + cid
    qbase = wid * _QPW
    idxb = (idxA, idxB)
    cwb = (cwA, cwB)
    outb = (outA, outB)
    rows = (rows0, rows1, rows2, rows3)
    gsem = (g0, g1, g2, g3)
    osem = (oA, oB)
    iisem = (iiA, iiB)
    icsem = (icA, icB)

    pltpu.sync_copy(idxs.at[qbase], idxA)
    pltpu.sync_copy(cws.at[qbase], cwA)
    pltpu.async_copy(idxs.at[qbase + 1], idxB, iiB)
    pltpu.async_copy(cws.at[qbase + 1], cwB, icB)
    pltpu.async_copy(table.at[idxA.at[0]], rows0, g0)
    pltpu.async_copy(table.at[idxA.at[1]], rows1, g1)

    def pair_body(i, carry):
        for par in range(2):
            q = qbase + 2 * i + par
            idx_v = idxb[par]
            cw_v = cwb[par]
            out_v = outb[par]
            oth = 1 - par
            for h in range(_H):
                if h == 0:
                    @pl.when(i > 0)
                    def _():
                        pltpu.make_async_copy(out_v, out.at[q],
                                              osem[par]).wait()
                if h == 6:
                    pltpu.make_async_copy(idxs.at[q], idxb[oth],
                                          iisem[oth]).wait()
                    pltpu.make_async_copy(cws.at[q], cwb[oth],
                                          icsem[oth]).wait()
                if h < 6:
                    nsrc = idx_v.at[h + 2]
                else:
                    nsrc = idxb[oth].at[h - 6]
                nb = (h + 2) % 4
                pltpu.async_copy(table.at[nsrc], rows[nb], gsem[nb])
                pltpu.make_async_copy(table.at[idx_v.at[0]], rows[h % 4],
                                      gsem[h % 4]).wait()
                buf = rows[h % 4]

                def s_body(jj, acc):
                    acc = list(acc)
                    wv = cw_v[h, jj]
                    for k in range(4):
                        j = jj * 4 + k
                        for c in range(4):
                            w = wv[4 * k + c]
                            acc[c] = acc[c] + w * buf[j, pl.ds(32 * c, 16)]
                            acc[4 + c] = (acc[4 + c]
                                          + w * buf[j, pl.ds(32 * c + 16, 16)])
                    return tuple(acc)

                zero = jnp.zeros((16,), jnp.float32)
                acc = lax.fori_loop(0, _SAMP // 4, s_body, (zero,) * 8)
                out_v[pl.ds(h * 32, 16)] = ((acc[0] + acc[1])
                                            + (acc[2] + acc[3]))
                out_v[pl.ds(h * 32 + 16, 16)] = ((acc[4] + acc[5])
                                                 + (acc[6] + acc[7]))
            pltpu.async_copy(out_v, out.at[q], osem[par])
            qn = jnp.minimum(q + 2, _NQ - 1)
            pltpu.async_copy(idxs.at[qn], idx_v, iisem[par])
            pltpu.async_copy(cws.at[qn], cw_v, icsem[par])
        return carry

    lax.fori_loop(0, _QPW // 2, pair_body, 0)

    pltpu.make_async_copy(table.at[idxA.at[0]], rows0, g0).wait()
    pltpu.make_async_copy(table.at[idxA.at[0]], rows1, g1).wait()
    pltpu.make_async_copy(idxs.at[qbase], idxB, iiB).wait()
    pltpu.make_async_copy(cws.at[qbase], cwB, icB).wait()
    pltpu.make_async_copy(outA, out.at[qbase], oA).wait()
    pltpu.make_async_copy(outB, out.at[qbase], oB).wait()


def _sc_gather(table, idxs, cws):
    mesh = plsc.VectorSubcoreMesh(core_axis_name="c", subcore_axis_name="s",
                                  num_cores=2, num_subcores=16)
    return pl.kernel(
        _sc_body,
        out_type=jax.ShapeDtypeStruct((_NQ, _EMBED), jnp.float32),
        mesh=mesh,
        scratch_types=[
            pltpu.VMEM((_H, _SAMP), jnp.int32),
            pltpu.VMEM((_H, _SAMP), jnp.int32),
            pltpu.VMEM((_H, _SAMP // 4, 16), jnp.float32),
            pltpu.VMEM((_H, _SAMP // 4, 16), jnp.float32),
            pltpu.VMEM((_SAMP, 128), jnp.float32),
            pltpu.VMEM((_SAMP, 128), jnp.float32),
            pltpu.VMEM((_SAMP, 128), jnp.float32),
            pltpu.VMEM((_SAMP, 128), jnp.float32),
            pltpu.VMEM((_EMBED,), jnp.float32),
            pltpu.VMEM((_EMBED,), jnp.float32),
            pltpu.SemaphoreType.DMA,
            pltpu.SemaphoreType.DMA,
            pltpu.SemaphoreType.DMA,
            pltpu.SemaphoreType.DMA,
            pltpu.SemaphoreType.DMA,
            pltpu.SemaphoreType.DMA,
            pltpu.SemaphoreType.DMA,
            pltpu.SemaphoreType.DMA,
            pltpu.SemaphoreType.DMA,
            pltpu.SemaphoreType.DMA,
        ],
    )(table, idxs, cws)


def _outproj_body(x_ref, w_ref, b_ref, out_ref):
    out_ref[...] = (
        jnp.dot(x_ref[...], w_ref[...], preferred_element_type=jnp.float32,
                precision=_PREC)
        + b_ref[...]
    )


def _outproj(x, W_out, b_out):
    rows = 512
    return pl.pallas_call(
        _outproj_body,
        grid=(_NQ // rows,),
        in_specs=[
            pl.BlockSpec((rows, _EMBED), lambda t: (t, 0)),
            pl.BlockSpec((_EMBED, _EMBED), lambda t: (0, 0)),
            pl.BlockSpec((1, _EMBED), lambda t: (0, 0)),
        ],
        out_specs=pl.BlockSpec((rows, _EMBED), lambda t: (t, 0)),
        out_shape=jax.ShapeDtypeStruct((_NQ, _EMBED), jnp.float32),
    )(x, W_out, b_out.reshape(1, _EMBED))


def _build_table(v4):
    parts = []
    off = 0
    for (hh, ww), npl in zip(_SS, _NP_L):
        hh, ww = int(hh), int(ww)
        w2 = ww + 2
        g = v4[:, :, off:off + hh * ww].reshape(_BS, _H, hh, ww, _HD)
        gp = jnp.pad(g, ((0, 0), (0, 0), (1, 1), (1, 1), (0, 0)))
        flat = gp.reshape(_BS, _H, npl, _HD)
        flat = jnp.pad(flat, ((0, 0), (0, 0), (0, w2 + 1), (0, 0)))
        quad = jnp.concatenate([
            flat[:, :, 0:npl],
            flat[:, :, 1:npl + 1],
            flat[:, :, w2:npl + w2],
            flat[:, :, w2 + 1:npl + w2 + 1],
        ], axis=-1)
        parts.append(quad)
        off += hh * ww
    table = jnp.concatenate(parts, axis=2)
    return table.reshape(_BS * _H * _PAD_ROWS, 4 * _HD)


def kernel(query, reference_points, value, value_spatial_shapes, W_off,
           b_off, W_attn, b_attn, W_val, b_val, W_out, b_out):
    v4 = _vproj(value, W_val, b_val)
    table = _build_table(v4)

    Wx = W_off.reshape(_EMBED, _TP, 2)[:, :, 0]
    Wy = W_off.reshape(_EMBED, _TP, 2)[:, :, 1]
    WQ = jnp.concatenate([Wx, Wy, W_attn], axis=1)
    bx = b_off.reshape(_TP, 2)[:, 0]
    by = b_off.reshape(_TP, 2)[:, 1]
    bq = jnp.concatenate([bx, by, b_attn])
    rp16 = reference_points.reshape(_BS, _LQ, 16)
    sloc2, aw, idx, cw4 = _prep(query, rp16, WQ, bq)

    idxs = idx.reshape(_NQ, _H, _SAMP)
    cws = jnp.moveaxis(cw4, 1, -1).reshape(_NQ, _H, _SAMP // 4, 16)

    core = _sc_gather(table, idxs, cws)

    out = _outproj(core, W_out, b_out).reshape(_BS, _LQ, _EMBED)

    sloc = jnp.stack([sloc2[:, 0], sloc2[:, 1]], axis=-1)
    sloc = sloc.reshape(_BS, _LQ, _H, _L, _P, 2)
    aw_out = aw.reshape(_BS, _LQ, _H, _L, _P)
    return (out, sloc, aw_out)


# speedup vs baseline: 3.0550x; 1.0940x over previous
import functools

import numpy as np
import jax
import jax.numpy as jnp
from jax import lax
from jax.experimental import pallas as pl
from jax.experimental.pallas import tpu as pltpu
from jax.experimental.pallas import tpu_sc as plsc

_EMBED = 256
_H = 8
_L = 4
_P = 16
_HD = 32
_BS = 4
_LQ = 1024
_SS = np.array([[64, 64], [32, 32], [16, 16], [8, 8]], dtype=np.int64)
_LV = int((_SS[:, 0] * _SS[:, 1]).sum())
_NP_L = [(int(h) + 2) * (int(w) + 2) for h, w in _SS]
_PAD_ROWS = int(sum(_NP_L))
_BASE_L = np.concatenate([[0], np.cumsum(_NP_L)[:-1]]).astype(np.int64)
_SAMP = _L * _P
_TP = _H * _L * _P
_NQ = _BS * _LQ
_NW = 32
_QPW = _NQ // _NW
_QT = 128
_PREC = lax.Precision.HIGHEST


def _build_consts():
    s = np.arange(_TP)
    lvl = (s // _P) % _L
    head = s // (_L * _P)
    gw = _SS[lvl, 1].astype(np.float32)
    gh = _SS[lvl, 0].astype(np.float32)
    stride = gw + 2.0
    sbase = (head * _PAD_ROWS + _BASE_L[lvl]).astype(np.float32)
    return np.stack([gw, gh, stride, sbase]).astype(np.float32)


def _build_sel():
    sel = np.zeros((16, 4 * _TP), np.float32)
    s = np.arange(_TP)
    lvl = (s // _P) % _L
    for comp in range(4):
        sel[lvl * 4 + comp, comp * _TP + s] = 1.0
    return sel


_CONSTS = _build_consts()
_SEL = _build_sel()


def _vproj_body(val_ref, w_ref, b_ref, out_ref):
    z = (jnp.dot(val_ref[0], w_ref[...], preferred_element_type=jnp.float32,
                 precision=_PREC)
         + b_ref[...])
    for h in range(_H):
        out_ref[0, h] = z[:, h * _HD:(h + 1) * _HD]


def _vproj(value, W_val, b_val):
    rows = 680
    return pl.pallas_call(
        _vproj_body,
        grid=(_BS, _LV // rows),
        in_specs=[
            pl.BlockSpec((1, rows, _EMBED), lambda b, t: (b, t, 0)),
            pl.BlockSpec((_EMBED, _EMBED), lambda b, t: (0, 0)),
            pl.BlockSpec((1, _EMBED), lambda b, t: (0, 0)),
        ],
        out_specs=pl.BlockSpec((1, _H, rows, _HD), lambda b, t: (b, 0, t, 0)),
        out_shape=jax.ShapeDtypeStruct((_BS, _H, _LV, _HD), jnp.float32),
    )(value, W_val, b_val.reshape(1, _EMBED))


def _prep_body(q_ref, rp_ref, wq_ref, bq_ref, sel_ref, cons_ref,
               sloc_ref, aw_ref, idx_ref, cw_ref):
    b = pl.program_id(0)
    q = q_ref[0]
    rp = rp_ref[0]
    z = (jnp.dot(q, wq_ref[...], preferred_element_type=jnp.float32,
                 precision=_PREC) + bq_ref[...])
    offx = z[:, :_TP]
    offy = z[:, _TP:2 * _TP]
    logits = z[:, 2 * _TP:]

    rsel = jnp.dot(rp, sel_ref[...], preferred_element_type=jnp.float32,
                   precision=_PREC)
    cx = rsel[:, :_TP]
    cy = rsel[:, _TP:2 * _TP]
    rw = rsel[:, 2 * _TP:3 * _TP]
    rh = rsel[:, 3 * _TP:]

    l3 = logits.reshape(_QT, _H, _SAMP)
    m = jnp.max(l3, axis=-1, keepdims=True)
    e = jnp.exp(l3 - m)
    aw = (e / jnp.sum(e, axis=-1, keepdims=True)).reshape(_QT, _TP)

    gw = cons_ref[0, :].reshape(1, _TP)
    gh = cons_ref[1, :].reshape(1, _TP)
    stride = cons_ref[2, :].reshape(1, _TP)
    sbase = cons_ref[3, :].reshape(1, _TP)

    scale = 0.5 / _P
    sx = cx + offx * (rw * scale)
    sy = cy + offy * (rh * scale)

    x = sx * gw - 0.5
    y = sy * gh - 0.5
    x0 = jnp.floor(x)
    y0 = jnp.floor(y)
    fx = x - x0
    fy = y - y0
    one = jnp.float32(1.0)
    vx0 = ((x0 >= 0) & (x0 <= gw - 1)).astype(jnp.float32)
    vx1 = ((x0 >= -1) & (x0 <= gw - 2)).astype(jnp.float32)
    vy0 = ((y0 >= 0) & (y0 <= gh - 1)).astype(jnp.float32)
    vy1 = ((y0 >= -1) & (y0 <= gh - 2)).astype(jnp.float32)

    cw00 = aw * (one - fx) * (one - fy) * vx0 * vy0
    cw01 = aw * fx * (one - fy) * vx1 * vy0
    cw10 = aw * (one - fx) * fy * vx0 * vy1
    cw11 = aw * fx * fy * vx1 * vy1

    x0c = jnp.clip(x0, -1.0, gw - 1)
    y0c = jnp.clip(y0, -1.0, gh - 1)
    ilocal = (y0c + 1.0) * stride + (x0c + 1.0)
    idx = (sbase + ilocal).astype(jnp.int32) + b * (_H * _PAD_ROWS)

    sloc_ref[0, 0] = sx
    sloc_ref[0, 1] = sy
    aw_ref[0] = aw
    idx_ref[0] = idx
    cw_ref[0, 0] = cw00
    cw_ref[0, 1] = cw01
    cw_ref[0, 2] = cw10
    cw_ref[0, 3] = cw11


def _prep(query, rp16, WQ, bq):
    grid = (_BS, _LQ // _QT)
    return pl.pallas_call(
        _prep_body,
        grid=grid,
        in_specs=[
            pl.BlockSpec((1, _QT, _EMBED), lambda b, t: (b, t, 0)),
            pl.BlockSpec((1, _QT, 16), lambda b, t: (b, t, 0)),
            pl.BlockSpec((_EMBED, 3 * _TP), lambda b, t: (0, 0)),
            pl.BlockSpec((1, 3 * _TP), lambda b, t: (0, 0)),
            pl.BlockSpec((16, 4 * _TP), lambda b, t: (0, 0)),
            pl.BlockSpec((4, _TP), lambda b, t: (0, 0)),
        ],
        out_specs=[
            pl.BlockSpec((1, 2, _QT, _TP), lambda b, t: (b, 0, t, 0)),
            pl.BlockSpec((1, _QT, _TP), lambda b, t: (b, t, 0)),
            pl.BlockSpec((1, _QT, _TP), lambda b, t: (b, t, 0)),
            pl.BlockSpec((1, 4, _QT, _TP), lambda b, t: (b, 0, t, 0)),
        ],
        out_shape=[
            jax.ShapeDtypeStruct((_BS, 2, _LQ, _TP), jnp.float32),
            jax.ShapeDtypeStruct((_BS, _LQ, _TP), jnp.float32),
            jax.ShapeDtypeStruct((_BS, _LQ, _TP), jnp.int32),
            jax.ShapeDtypeStruct((_BS, 4, _LQ, _TP), jnp.float32),
        ],
    )(query, rp16, WQ, bq.reshape(1, 3 * _TP),
      jnp.asarray(_SEL), jnp.asarray(_CONSTS))


def _sc_body(table, idxs, cws, out,
             idxA, idxB, cwA, cwB, rows0, rows1, rows2, rows3, outA, outB,
             g0, g1, g2, g3, oA, oB, iiA, iiB, icA, icB):
    cid = lax.axis_index("c")
    sid = lax.axis_index("s")
    wid = sid * 2 + cid
    qbase = wid * _QPW
    idxb = (idxA, idxB)
    cwb = (cwA, cwB)
    outb = (outA, outB)
    rows = (rows0, rows1, rows2, rows3)
    gsem = (g0, g1, g2, g3)
    osem = (oA, oB)
    iisem = (iiA, iiB)
    icsem = (icA, icB)

    pltpu.sync_copy(idxs.at[qbase], idxA)
    pltpu.sync_copy(cws.at[qbase], cwA)
    pltpu.async_copy(idxs.at[qbase + 1], idxB, iiB)
    pltpu.async_copy(cws.at[qbase + 1], cwB, icB)
    pltpu.async_copy(table.at[idxA.at[0]], rows0, g0)
    pltpu.async_copy(table.at[idxA.at[1]], rows1, g1)

    def pair_body(i, carry):
        for par in range(2):
            q = qbase + 2 * i + par
            idx_v = idxb[par]
            cw_v = cwb[par]
            out_v = outb[par]
            oth = 1 - par
            for h in range(_H):
                if h == 0:
                    @pl.when(i > 0)
                    def _():
                        pltpu.make_async_copy(out_v, out.at[q],
                                              osem[par]).wait()
                if h == 6:
                    pltpu.make_async_copy(idxs.at[q], idxb[oth],
                                          iisem[oth]).wait()
                    pltpu.make_async_copy(cws.at[q], cwb[oth],
                                          icsem[oth]).wait()
                if h < 6:
                    nsrc = idx_v.at[h + 2]
                else:
                    nsrc = idxb[oth].at[h - 6]
                nb = (h + 2) % 4
                pltpu.async_copy(table.at[nsrc], rows[nb], gsem[nb])
                pltpu.make_async_copy(table.at[idx_v.at[0]], rows[h % 4],
                                      gsem[h % 4]).wait()
                buf = rows[h % 4]

                def s_body(jj, acc):
                    acc = list(acc)
                    wv = cw_v[h, jj]
                    for k in range(4):
                        j = jj * 4 + k
                        for c in range(4):
                            w = wv[4 * k + c]
                            acc[c] = acc[c] + w * buf[j, pl.ds(32 * c, 16)]
                            acc[4 + c] = (acc[4 + c]
                                          + w * buf[j, pl.ds(32 * c + 16, 16)])
                    return tuple(acc)

                zero = jnp.zeros((16,), jnp.float32)
                acc = lax.fori_loop(0, _SAMP // 4, s_body, (zero,) * 8)
                out_v[pl.ds(h * 32, 16)] = ((acc[0] + acc[1])
                                            + (acc[2] + acc[3]))
                out_v[pl.ds(h * 32 + 16, 16)] = ((acc[4] + acc[5])
                                                 + (acc[6] + acc[7]))
            pltpu.async_copy(out_v, out.at[q], osem[par])
            qn = jnp.minimum(q + 2, _NQ - 1)
            pltpu.async_copy(idxs.at[qn], idx_v, iisem[par])
            pltpu.async_copy(cws.at[qn], cw_v, icsem[par])
        return carry

    lax.fori_loop(0, _QPW // 2, pair_body, 0)

    pltpu.make_async_copy(table.at[idxA.at[0]], rows0, g0).wait()
    pltpu.make_async_copy(table.at[idxA.at[0]], rows1, g1).wait()
    pltpu.make_async_copy(idxs.at[qbase], idxB, iiB).wait()
    pltpu.make_async_copy(cws.at[qbase], cwB, icB).wait()
    pltpu.make_async_copy(outA, out.at[qbase], oA).wait()
    pltpu.make_async_copy(outB, out.at[qbase], oB).wait()


def _sc_gather(table, idxs, cws):
    mesh = plsc.VectorSubcoreMesh(core_axis_name="c", subcore_axis_name="s",
                                  num_cores=2, num_subcores=16)
    return pl.kernel(
        _sc_body,
        out_type=jax.ShapeDtypeStruct((_NQ, _EMBED), jnp.float32),
        mesh=mesh,
        scratch_types=[
            pltpu.VMEM((_H, _SAMP), jnp.int32),
            pltpu.VMEM((_H, _SAMP), jnp.int32),
            pltpu.VMEM((_H, _SAMP // 4, 16), jnp.float32),
            pltpu.VMEM((_H, _SAMP // 4, 16), jnp.float32),
            pltpu.VMEM((_SAMP, 128), jnp.float32),
            pltpu.VMEM((_SAMP, 128), jnp.float32),
            pltpu.VMEM((_SAMP, 128), jnp.float32),
            pltpu.VMEM((_SAMP, 128), jnp.float32),
            pltpu.VMEM((_EMBED,), jnp.float32),
            pltpu.VMEM((_EMBED,), jnp.float32),
            pltpu.SemaphoreType.DMA,
            pltpu.SemaphoreType.DMA,
            pltpu.SemaphoreType.DMA,
            pltpu.SemaphoreType.DMA,
            pltpu.SemaphoreType.DMA,
            pltpu.SemaphoreType.DMA,
            pltpu.SemaphoreType.DMA,
            pltpu.SemaphoreType.DMA,
            pltpu.SemaphoreType.DMA,
            pltpu.SemaphoreType.DMA,
        ],
    )(table, idxs, cws)


def _outproj_body(x_ref, w_ref, b_ref, out_ref):
    out_ref[...] = (
        jnp.dot(x_ref[...], w_ref[...], preferred_element_type=jnp.float32,
                precision=_PREC)
        + b_ref[...]
    )


def _outproj(x, W_out, b_out):
    rows = 512
    return pl.pallas_call(
        _outproj_body,
        grid=(_NQ // rows,),
        in_specs=[
            pl.BlockSpec((rows, _EMBED), lambda t: (t, 0)),
            pl.BlockSpec((_EMBED, _EMBED), lambda t: (0, 0)),
            pl.BlockSpec((1, _EMBED), lambda t: (0, 0)),
        ],
        out_specs=pl.BlockSpec((rows, _EMBED), lambda t: (t, 0)),
        out_shape=jax.ShapeDtypeStruct((_NQ, _EMBED), jnp.float32),
    )(x, W_out, b_out.reshape(1, _EMBED))


_NPE_L = [npl + int(w) + 3 for npl, (h, w) in zip(_NP_L, _SS)]


def _quadify_body(pe0, pe1, pe2, pe3, out_ref):
    pes = (pe0, pe1, pe2, pe3)
    base = 0
    for lvl, ((hh, ww), npl) in enumerate(zip(_SS, _NP_L)):
        w2 = int(ww) + 2
        pf = pes[lvl][0, 0]
        quad = jnp.concatenate([
            pf[0:npl],
            pf[1:npl + 1],
            pf[w2:npl + w2],
            pf[w2 + 1:npl + w2 + 1],
        ], axis=-1)
        out_ref[0, 0, base:base + npl] = quad
        base += npl


def _build_table(v4):
    pes = []
    off = 0
    for (hh, ww), npl, npe in zip(_SS, _NP_L, _NPE_L):
        hh, ww = int(hh), int(ww)
        g = v4[:, :, off:off + hh * ww].reshape(_BS, _H, hh, ww, _HD)
        gp = jnp.pad(g, ((0, 0), (0, 0), (1, 1), (1, 1), (0, 0)))
        flat = gp.reshape(_BS, _H, npl, _HD)
        pes.append(jnp.pad(flat, ((0, 0), (0, 0), (0, npe - npl), (0, 0))))
        off += hh * ww
    table = pl.pallas_call(
        _quadify_body,
        grid=(_BS, _H),
        in_specs=[
            pl.BlockSpec((1, 1, npe, _HD), lambda b, h: (b, h, 0, 0))
            for npe in _NPE_L
        ],
        out_specs=pl.BlockSpec((1, 1, _PAD_ROWS, 4 * _HD),
                               lambda b, h: (b, h, 0, 0)),
        out_shape=jax.ShapeDtypeStruct((_BS, _H, _PAD_ROWS, 4 * _HD),
                                       jnp.float32),
    )(*pes)
    return table.reshape(_BS * _H * _PAD_ROWS, 4 * _HD)


def kernel(query, reference_points, value, value_spatial_shapes, W_off,
           b_off, W_attn, b_attn, W_val, b_val, W_out, b_out):
    v4 = _vproj(value, W_val, b_val)
    table = _build_table(v4)

    Wx = W_off.reshape(_EMBED, _TP, 2)[:, :, 0]
    Wy = W_off.reshape(_EMBED, _TP, 2)[:, :, 1]
    WQ = jnp.concatenate([Wx, Wy, W_attn], axis=1)
    bx = b_off.reshape(_TP, 2)[:, 0]
    by = b_off.reshape(_TP, 2)[:, 1]
    bq = jnp.concatenate([bx, by, b_attn])
    rp16 = reference_points.reshape(_BS, _LQ, 16)
    sloc2, aw, idx, cw4 = _prep(query, rp16, WQ, bq)

    idxs = idx.reshape(_NQ, _H, _SAMP)
    cws = jnp.moveaxis(cw4, 1, -1).reshape(_NQ, _H, _SAMP // 4, 16)

    core = _sc_gather(table, idxs, cws)

    out = _outproj(core, W_out, b_out).reshape(_BS, _LQ, _EMBED)

    sloc = jnp.stack([sloc2[:, 0], sloc2[:, 1]], axis=-1)
    sloc = sloc.reshape(_BS, _LQ, _H, _L, _P, 2)
    aw_out = aw.reshape(_BS, _LQ, _H, _L, _P)
    return (out, sloc, aw_out)
